# baseline (speedup 1.0000x reference)
import functools
import math

import jax
import jax.numpy as jnp
from jax import lax
from jax.experimental import pallas as pl
from jax.experimental.pallas import tpu as pltpu
from jax.experimental.pallas import tpu_sc as plsc

TILE_R = 128
_NC = 2
_NS = 16
_L = 16
_NW = _NC * _NS


def _router_body(x_ref, wr_ref, br_ref,
                 e0_ref, e1_ref, r0_ref, r1_ref, w0_ref, w1_ref, cnt_ref,
                 carry_ref):
    t = pl.program_id(0)
    nt = pl.num_programs(0)

    @pl.when(t == 0)
    def _():
        carry_ref[...] = jnp.zeros_like(carry_ref)

    xb = x_ref[...]
    logits = jnp.dot(xb, wr_ref[...], preferred_element_type=jnp.float32)
    logits = logits + br_ref[...]
    m = jnp.max(logits, axis=-1, keepdims=True)
    p = jnp.exp(logits - m)
    p = p / jnp.sum(p, axis=-1, keepdims=True)
    BT, E = p.shape
    idx = jax.lax.broadcasted_iota(jnp.int32, p.shape, 1)
    p0 = jnp.max(p, axis=-1, keepdims=True)
    e0 = jnp.min(jnp.where(p == p0, idx, E), axis=-1, keepdims=True)
    oh0 = (idx == e0)
    pm = jnp.where(oh0, -1.0, p)
    p1 = jnp.max(pm, axis=-1, keepdims=True)
    e1 = jnp.min(jnp.where(pm == p1, idx, E), axis=-1, keepdims=True)
    oh1 = (idx == e1)
    s = jnp.maximum(p0 + p1, 1e-9)

    oh0f = oh0.astype(jnp.float32)
    oh1f = oh1.astype(jnp.float32)
    h = oh0f + oh1f
    ri = jax.lax.broadcasted_iota(jnp.int32, (BT, BT), 0)
    ci = jax.lax.broadcasted_iota(jnp.int32, (BT, BT), 1)
    tril = (ci < ri).astype(jnp.float32)
    C = jnp.dot(tril, h, preferred_element_type=jnp.float32)
    carry = carry_ref[0:1, :]
    Cg = C + carry
    r0 = jnp.sum(oh0f * Cg, axis=-1, keepdims=True)
    r1 = jnp.sum(oh1f * Cg, axis=-1, keepdims=True)

    e0_ref[...] = e0.reshape(1, 1, BT)
    e1_ref[...] = e1.reshape(1, 1, BT)
    r0_ref[...] = r0.astype(jnp.int32).reshape(1, 1, BT)
    r1_ref[...] = r1.astype(jnp.int32).reshape(1, 1, BT)
    w0_ref[...] = (p0 / s).reshape(1, 1, BT)
    w1_ref[...] = (p1 / s).reshape(1, 1, BT)

    new_carry = carry + jnp.sum(h, axis=0, keepdims=True)
    carry_ref[0:1, :] = new_carry

    @pl.when(t == nt - 1)
    def _():
        cnt_ref[...] = new_carry.astype(jnp.int32)


def _mlp_body(gid_ref, xs_ref, w1_ref, b1_ref, w2_ref, b2_ref, out_ref):
    xb = xs_ref[...].astype(jnp.bfloat16)
    h = jnp.dot(xb, w1_ref[0], preferred_element_type=jnp.float32) + b1_ref[0]
    h = 0.5 * h * (1.0 + jax.lax.erf(h * (1.0 / math.sqrt(2.0))))
    out_ref[...] = jnp.dot(h.astype(jnp.bfloat16), w2_ref[0],
                           preferred_element_type=jnp.float32) + b2_ref[0]


def _vgather(vec, idx):
    dnums = lax.GatherDimensionNumbers(
        offset_dims=(), collapsed_slice_dims=(0,), start_index_map=(0,))
    return lax.gather(vec, idx[:, None], dnums, (1,),
                      mode=lax.GatherScatterMode.PROMISE_IN_BOUNDS)

def _sc_dispatch_body(T, D, E, CH,
                      x_hbm, e0_hbm, e1_hbm, r0_hbm, r1_hbm, off_hbm, xs_hbm,
                      xv, e0v, e1v, r0v, r1v, offv, idx0, idx1, sem):
    tpw = T // _NW
    wid = lax.axis_index("s") * _NC + lax.axis_index("c")
    t0 = pl.multiple_of(wid * tpw, tpw)
    pltpu.sync_copy(e0_hbm.at[pl.ds(t0, tpw)], e0v)
    pltpu.sync_copy(e1_hbm.at[pl.ds(t0, tpw)], e1v)
    pltpu.sync_copy(r0_hbm.at[pl.ds(t0, tpw)], r0v)
    pltpu.sync_copy(r1_hbm.at[pl.ds(t0, tpw)], r1v)
    pltpu.sync_copy(off_hbm, offv)
    off_vec = offv[...]

    def chunk(c, carry):
        cb = pl.multiple_of(c * CH, CH)
        pltpu.sync_copy(x_hbm.at[pl.ds(t0 + cb, CH)], xv)
        for g in range(CH // _L):
            o = pl.multiple_of(cb + g * _L, _L)
            pos0 = _vgather(off_vec, e0v[pl.ds(o, _L)]) + r0v[pl.ds(o, _L)]
            idx0[pl.ds(g * _L, _L)] = pos0
            pos1 = _vgather(off_vec, e1v[pl.ds(o, _L)]) + r1v[pl.ds(o, _L)]
            idx1[pl.ds(g * _L, _L)] = pos1
        cp0 = pltpu.async_copy(xv, xs_hbm.at[idx0], sem)
        cp1 = pltpu.async_copy(xv, xs_hbm.at[idx1], sem)
        cp0.wait()
        cp1.wait()
        return carry

    lax.fori_loop(0, tpw // CH, chunk, 0)


def _sc_combine_body(T, D, E, CH,
                     x_hbm, ys_hbm, e0_hbm, e1_hbm, r0_hbm, r1_hbm,
                     w0_hbm, w1_hbm, off_hbm, y_hbm,
                     xv, av, bv, e0v, e1v, r0v, r1v, w0v, w1v, offv,
                     idx0, idx1, sem):
    tpw = T // _NW
    wid = lax.axis_index("s") * _NC + lax.axis_index("c")
    t0 = pl.multiple_of(wid * tpw, tpw)
    pltpu.sync_copy(e0_hbm.at[pl.ds(t0, tpw)], e0v)
    pltpu.sync_copy(e1_hbm.at[pl.ds(t0, tpw)], e1v)
    pltpu.sync_copy(r0_hbm.at[pl.ds(t0, tpw)], r0v)
    pltpu.sync_copy(r1_hbm.at[pl.ds(t0, tpw)], r1v)
    pltpu.sync_copy(w0_hbm.at[pl.ds(t0, tpw)], w0v)
    pltpu.sync_copy(w1_hbm.at[pl.ds(t0, tpw)], w1v)
    pltpu.sync_copy(off_hbm, offv)
    off_vec = offv[...]

    def chunk(c, carry):
        cb = pl.multiple_of(c * CH, CH)
        pos0 = _vgather(off_vec, e0v[pl.ds(cb, _L)]) + r0v[pl.ds(cb, _L)]
        idx0[...] = pos0
        pos1 = _vgather(off_vec, e1v[pl.ds(cb, _L)]) + r1v[pl.ds(cb, _L)]
        idx1[...] = pos1
        cpa = pltpu.async_copy(ys_hbm.at[idx0], av, sem)
        cpb = pltpu.async_copy(ys_hbm.at[idx1], bv, sem)
        pltpu.sync_copy(x_hbm.at[pl.ds(t0 + cb, CH)], xv)
        cpa.wait()
        cpb.wait()
        w0c = w0v[pl.ds(cb, _L)]
        w1c = w1v[pl.ds(cb, _L)]
        for j in range(CH):
            jv = jnp.full((_L,), j, jnp.int32)
            wa = _vgather(w0c, jv)
            wb = _vgather(w1c, jv)

            def vg(v, cc, j=j, wa=wa, wb=wb):
                sl = pl.ds(pl.multiple_of(v * _L, _L), _L)
                xv[j, sl] = xv[j, sl] + wa * av[j, sl] + wb * bv[j, sl]
                return cc

            lax.fori_loop(0, D // _L, vg, 0)
        pltpu.sync_copy(xv, y_hbm.at[pl.ds(t0 + cb, CH)])
        return carry

    lax.fori_loop(0, tpw // CH, chunk, 0)


def kernel(x, Wr, br, W1, b1, W2, b2):
    T, D = x.shape
    E = Wr.shape[1]
    H = W1.shape[2]
    BT_R = 1024
    nblk = T // BT_R

    outs = pl.pallas_call(
        _router_body,
        grid=(nblk,),
        in_specs=[
            pl.BlockSpec((BT_R, D), lambda t: (t, 0)),
            pl.BlockSpec((D, E), lambda t: (0, 0)),
            pl.BlockSpec((E,), lambda t: (0,)),
        ],
        out_specs=[
            pl.BlockSpec((1, 1, BT_R), lambda t: (t, 0, 0)),
            pl.BlockSpec((1, 1, BT_R), lambda t: (t, 0, 0)),
            pl.BlockSpec((1, 1, BT_R), lambda t: (t, 0, 0)),
            pl.BlockSpec((1, 1, BT_R), lambda t: (t, 0, 0)),
            pl.BlockSpec((1, 1, BT_R), lambda t: (t, 0, 0)),
            pl.BlockSpec((1, 1, BT_R), lambda t: (t, 0, 0)),
            pl.BlockSpec((1, E), lambda t: (0, 0)),
        ],
        out_shape=[
            jax.ShapeDtypeStruct((nblk, 1, BT_R), jnp.int32),
            jax.ShapeDtypeStruct((nblk, 1, BT_R), jnp.int32),
            jax.ShapeDtypeStruct((nblk, 1, BT_R), jnp.int32),
            jax.ShapeDtypeStruct((nblk, 1, BT_R), jnp.int32),
            jax.ShapeDtypeStruct((nblk, 1, BT_R), jnp.float32),
            jax.ShapeDtypeStruct((nblk, 1, BT_R), jnp.float32),
            jax.ShapeDtypeStruct((1, E), jnp.int32),
        ],
        scratch_shapes=[pltpu.VMEM((8, E), jnp.float32)],
    )(x, Wr, br)
    e0, e1, r0, r1, w0, w1, counts = outs
    e0 = e0.reshape(T)
    e1 = e1.reshape(T)
    r0 = r0.reshape(T)
    r1 = r1.reshape(T)
    w0 = w0.reshape(T)
    w1 = w1.reshape(T)
    counts = counts.reshape(E)

    padded = ((counts + TILE_R - 1) // TILE_R) * TILE_R
    ends = jnp.cumsum(padded)
    off = (ends - padded).astype(jnp.int32)
    PADDED = T * 2 + E * TILE_R
    NT = PADDED // TILE_R
    tile_starts = jnp.arange(NT, dtype=jnp.int32) * TILE_R
    gids = jnp.clip(jnp.searchsorted(ends, tile_starts, side="right"),
                    0, E - 1).astype(jnp.int32)

    mesh = plsc.VectorSubcoreMesh(core_axis_name="c", subcore_axis_name="s")
    CH_A = 32
    dispatch = pl.kernel(
        functools.partial(_sc_dispatch_body, T, D, E, CH_A),
        mesh=mesh,
        out_type=jax.ShapeDtypeStruct((PADDED, D), jnp.float32),
        scratch_types=[
            pltpu.VMEM((CH_A, D), jnp.float32),
            pltpu.VMEM((T // _NW,), jnp.int32),
            pltpu.VMEM((T // _NW,), jnp.int32),
            pltpu.VMEM((T // _NW,), jnp.int32),
            pltpu.VMEM((T // _NW,), jnp.int32),
            pltpu.VMEM((E,), jnp.int32),
            pltpu.VMEM((CH_A,), jnp.int32),
            pltpu.VMEM((CH_A,), jnp.int32),
            pltpu.SemaphoreType.DMA,
        ],
    )
    xs = dispatch(x, e0, e1, r0, r1, off)

    b1r = b1.reshape(E, 1, H)
    b2r = b2.reshape(E, 1, D)
    W1b = W1.astype(jnp.bfloat16)
    W2b = W2.astype(jnp.bfloat16)
    grid_spec = pltpu.PrefetchScalarGridSpec(
        num_scalar_prefetch=1,
        grid=(NT,),
        in_specs=[
            pl.BlockSpec((TILE_R, D), lambda i, g: (i, 0)),
            pl.BlockSpec((1, D, H), lambda i, g: (g[i], 0, 0)),
            pl.BlockSpec((1, 1, H), lambda i, g: (g[i], 0, 0)),
            pl.BlockSpec((1, H, D), lambda i, g: (g[i], 0, 0)),
            pl.BlockSpec((1, 1, D), lambda i, g: (g[i], 0, 0)),
        ],
        out_specs=pl.BlockSpec((TILE_R, D), lambda i, g: (i, 0)),
    )
    ys = pl.pallas_call(
        _mlp_body,
        grid_spec=grid_spec,
        out_shape=jax.ShapeDtypeStruct((PADDED, D), jnp.float32),
    )(gids, xs, W1b, b1r, W2b, b2r)

    CH_B = 16
    combine = pl.kernel(
        functools.partial(_sc_combine_body, T, D, E, CH_B),
        mesh=mesh,
        out_type=jax.ShapeDtypeStruct((T, D), jnp.float32),
        scratch_types=[
            pltpu.VMEM((CH_B, D), jnp.float32),
            pltpu.VMEM((CH_B, D), jnp.float32),
            pltpu.VMEM((CH_B, D), jnp.float32),
            pltpu.VMEM((T // _NW,), jnp.int32),
            pltpu.VMEM((T // _NW,), jnp.int32),
            pltpu.VMEM((T // _NW,), jnp.int32),
            pltpu.VMEM((T // _NW,), jnp.int32),
            pltpu.VMEM((T // _NW,), jnp.float32),
            pltpu.VMEM((T // _NW,), jnp.float32),
            pltpu.VMEM((E,), jnp.int32),
            pltpu.VMEM((_L,), jnp.int32),
            pltpu.VMEM((_L,), jnp.int32),
            pltpu.SemaphoreType.DMA,
        ],
    )
    y = combine(x, ys, e0, e1, r0, r1, w0, w1, off)
    return y


# speedup vs baseline: 1.3484x; 1.3484x over previous
import functools
import math

import jax
import jax.numpy as jnp
from jax import lax
from jax.experimental import pallas as pl
from jax.experimental.pallas import tpu as pltpu
from jax.experimental.pallas import tpu_sc as plsc

TILE_R = 128
_NC = 2
_NS = 16
_L = 16
_NW = _NC * _NS


def _router_body(x_ref, wr_ref, br_ref,
                 e0_ref, e1_ref, r0_ref, r1_ref, w0_ref, w1_ref, cnt_ref,
                 carry_ref):
    t = pl.program_id(0)
    nt = pl.num_programs(0)

    @pl.when(t == 0)
    def _():
        carry_ref[...] = jnp.zeros_like(carry_ref)

    xb = x_ref[...]
    logits = jnp.dot(xb, wr_ref[...], preferred_element_type=jnp.float32)
    logits = logits + br_ref[...]
    m = jnp.max(logits, axis=-1, keepdims=True)
    p = jnp.exp(logits - m)
    p = p / jnp.sum(p, axis=-1, keepdims=True)
    BT, E = p.shape
    idx = jax.lax.broadcasted_iota(jnp.int32, p.shape, 1)
    p0 = jnp.max(p, axis=-1, keepdims=True)
    e0 = jnp.min(jnp.where(p == p0, idx, E), axis=-1, keepdims=True)
    oh0 = (idx == e0)
    pm = jnp.where(oh0, -1.0, p)
    p1 = jnp.max(pm, axis=-1, keepdims=True)
    e1 = jnp.min(jnp.where(pm == p1, idx, E), axis=-1, keepdims=True)
    oh1 = (idx == e1)
    s = jnp.maximum(p0 + p1, 1e-9)

    oh0f = oh0.astype(jnp.float32)
    oh1f = oh1.astype(jnp.float32)
    h = oh0f + oh1f
    ri = jax.lax.broadcasted_iota(jnp.int32, (BT, BT), 0)
    ci = jax.lax.broadcasted_iota(jnp.int32, (BT, BT), 1)
    tril = (ci < ri).astype(jnp.float32)
    C = jnp.dot(tril, h, preferred_element_type=jnp.float32)
    carry = carry_ref[0:1, :]
    Cg = C + carry
    r0 = jnp.sum(oh0f * Cg, axis=-1, keepdims=True)
    r1 = jnp.sum(oh1f * Cg, axis=-1, keepdims=True)

    e0_ref[...] = e0.reshape(1, 1, BT)
    e1_ref[...] = e1.reshape(1, 1, BT)
    r0_ref[...] = r0.astype(jnp.int32).reshape(1, 1, BT)
    r1_ref[...] = r1.astype(jnp.int32).reshape(1, 1, BT)
    w0_ref[...] = (p0 / s).reshape(1, 1, BT)
    w1_ref[...] = (p1 / s).reshape(1, 1, BT)

    new_carry = carry + jnp.sum(h, axis=0, keepdims=True)
    carry_ref[0:1, :] = new_carry

    @pl.when(t == nt - 1)
    def _():
        cnt_ref[...] = new_carry.astype(jnp.int32)


def _mlp_body(gid_ref, xs_ref, w1_ref, b1_ref, w2_ref, b2_ref, out_ref):
    xb = xs_ref[...]
    h = jnp.dot(xb, w1_ref[0], preferred_element_type=jnp.float32) + b1_ref[0]
    h = 0.5 * h * (1.0 + jax.lax.erf(h * (1.0 / math.sqrt(2.0))))
    out_ref[...] = jnp.dot(h, w2_ref[0],
                           preferred_element_type=jnp.float32) + b2_ref[0]


def _vgather(vec, idx):
    dnums = lax.GatherDimensionNumbers(
        offset_dims=(), collapsed_slice_dims=(0,), start_index_map=(0,))
    return lax.gather(vec, idx[:, None], dnums, (1,),
                      mode=lax.GatherScatterMode.PROMISE_IN_BOUNDS)

def _sc_dispatch_body(T, D, E, CH,
                      x_hbm, e0_hbm, e1_hbm, r0_hbm, r1_hbm, off_hbm, xs_hbm,
                      xv, e0v, e1v, r0v, r1v, offv, idx0, idx1, sem):
    tpw = T // _NW
    wid = lax.axis_index("s") * _NC + lax.axis_index("c")
    t0 = pl.multiple_of(wid * tpw, tpw)
    pltpu.sync_copy(e0_hbm.at[pl.ds(t0, tpw)], e0v)
    pltpu.sync_copy(e1_hbm.at[pl.ds(t0, tpw)], e1v)
    pltpu.sync_copy(r0_hbm.at[pl.ds(t0, tpw)], r0v)
    pltpu.sync_copy(r1_hbm.at[pl.ds(t0, tpw)], r1v)
    pltpu.sync_copy(off_hbm, offv)
    off_vec = offv[...]

    def chunk(c, carry):
        cb = pl.multiple_of(c * CH, CH)
        pltpu.sync_copy(x_hbm.at[pl.ds(t0 + cb, CH)], xv)
        for g in range(CH // _L):
            o = pl.multiple_of(cb + g * _L, _L)
            pos0 = _vgather(off_vec, e0v[pl.ds(o, _L)]) + r0v[pl.ds(o, _L)]
            idx0[pl.ds(g * _L, _L)] = pos0
            pos1 = _vgather(off_vec, e1v[pl.ds(o, _L)]) + r1v[pl.ds(o, _L)]
            idx1[pl.ds(g * _L, _L)] = pos1
        cp0 = pltpu.async_copy(xv, xs_hbm.at[idx0], sem)
        cp1 = pltpu.async_copy(xv, xs_hbm.at[idx1], sem)
        cp0.wait()
        cp1.wait()
        return carry

    lax.fori_loop(0, tpw // CH, chunk, 0)


def _sc_combine_body(T, D, E, CH,
                     x_hbm, ys_hbm, e0_hbm, e1_hbm, r0_hbm, r1_hbm,
                     w0_hbm, w1_hbm, off_hbm, y_hbm,
                     xv, av, bv, e0v, e1v, r0v, r1v, w0v, w1v, offv,
                     idx0, idx1, sem):
    tpw = T // _NW
    wid = lax.axis_index("s") * _NC + lax.axis_index("c")
    t0 = pl.multiple_of(wid * tpw, tpw)
    pltpu.sync_copy(e0_hbm.at[pl.ds(t0, tpw)], e0v)
    pltpu.sync_copy(e1_hbm.at[pl.ds(t0, tpw)], e1v)
    pltpu.sync_copy(r0_hbm.at[pl.ds(t0, tpw)], r0v)
    pltpu.sync_copy(r1_hbm.at[pl.ds(t0, tpw)], r1v)
    pltpu.sync_copy(w0_hbm.at[pl.ds(t0, tpw)], w0v)
    pltpu.sync_copy(w1_hbm.at[pl.ds(t0, tpw)], w1v)
    pltpu.sync_copy(off_hbm, offv)
    off_vec = offv[...]

    def chunk(c, carry):
        cb = pl.multiple_of(c * CH, CH)
        pos0 = _vgather(off_vec, e0v[pl.ds(cb, _L)]) + r0v[pl.ds(cb, _L)]
        idx0[...] = pos0
        pos1 = _vgather(off_vec, e1v[pl.ds(cb, _L)]) + r1v[pl.ds(cb, _L)]
        idx1[...] = pos1
        cpa = pltpu.async_copy(ys_hbm.at[idx0], av, sem)
        cpb = pltpu.async_copy(ys_hbm.at[idx1], bv, sem)
        pltpu.sync_copy(x_hbm.at[pl.ds(t0 + cb, CH)], xv)
        cpa.wait()
        cpb.wait()
        w0c = w0v[pl.ds(cb, _L)]
        w1c = w1v[pl.ds(cb, _L)]
        for j in range(CH):
            jv = jnp.full((_L,), j, jnp.int32)
            wa = _vgather(w0c, jv)
            wb = _vgather(w1c, jv)

            UN = 8

            def vg(v, cc, j=j, wa=wa, wb=wb):
                for u in range(UN):
                    sl = pl.ds(pl.multiple_of(v * (_L * UN), _L) + u * _L, _L)
                    xv[j, sl] = xv[j, sl] + wa * av[j, sl] + wb * bv[j, sl]
                return cc

            lax.fori_loop(0, D // (_L * UN), vg, 0)
        pltpu.sync_copy(xv, y_hbm.at[pl.ds(t0 + cb, CH)])
        return carry

    lax.fori_loop(0, tpw // CH, chunk, 0)


def kernel(x, Wr, br, W1, b1, W2, b2):
    T, D = x.shape
    E = Wr.shape[1]
    H = W1.shape[2]
    BT_R = 1024
    nblk = T // BT_R

    outs = pl.pallas_call(
        _router_body,
        grid=(nblk,),
        in_specs=[
            pl.BlockSpec((BT_R, D), lambda t: (t, 0)),
            pl.BlockSpec((D, E), lambda t: (0, 0)),
            pl.BlockSpec((E,), lambda t: (0,)),
        ],
        out_specs=[
            pl.BlockSpec((1, 1, BT_R), lambda t: (t, 0, 0)),
            pl.BlockSpec((1, 1, BT_R), lambda t: (t, 0, 0)),
            pl.BlockSpec((1, 1, BT_R), lambda t: (t, 0, 0)),
            pl.BlockSpec((1, 1, BT_R), lambda t: (t, 0, 0)),
            pl.BlockSpec((1, 1, BT_R), lambda t: (t, 0, 0)),
            pl.BlockSpec((1, 1, BT_R), lambda t: (t, 0, 0)),
            pl.BlockSpec((1, E), lambda t: (0, 0)),
        ],
        out_shape=[
            jax.ShapeDtypeStruct((nblk, 1, BT_R), jnp.int32),
            jax.ShapeDtypeStruct((nblk, 1, BT_R), jnp.int32),
            jax.ShapeDtypeStruct((nblk, 1, BT_R), jnp.int32),
            jax.ShapeDtypeStruct((nblk, 1, BT_R), jnp.int32),
            jax.ShapeDtypeStruct((nblk, 1, BT_R), jnp.float32),
            jax.ShapeDtypeStruct((nblk, 1, BT_R), jnp.float32),
            jax.ShapeDtypeStruct((1, E), jnp.int32),
        ],
        scratch_shapes=[pltpu.VMEM((8, E), jnp.float32)],
    )(x, Wr, br)
    e0, e1, r0, r1, w0, w1, counts = outs
    e0 = e0.reshape(T)
    e1 = e1.reshape(T)
    r0 = r0.reshape(T)
    r1 = r1.reshape(T)
    w0 = w0.reshape(T)
    w1 = w1.reshape(T)
    counts = counts.reshape(E)

    padded = ((counts + TILE_R - 1) // TILE_R) * TILE_R
    ends = jnp.cumsum(padded)
    off = (ends - padded).astype(jnp.int32)
    PADDED = T * 2 + E * TILE_R
    NT = PADDED // TILE_R
    tile_starts = jnp.arange(NT, dtype=jnp.int32) * TILE_R
    gids = jnp.clip(jnp.searchsorted(ends, tile_starts, side="right"),
                    0, E - 1).astype(jnp.int32)

    mesh = plsc.VectorSubcoreMesh(core_axis_name="c", subcore_axis_name="s")
    CH_A = 32
    dispatch = pl.kernel(
        functools.partial(_sc_dispatch_body, T, D, E, CH_A),
        mesh=mesh,
        out_type=jax.ShapeDtypeStruct((PADDED, D), jnp.float32),
        scratch_types=[
            pltpu.VMEM((CH_A, D), jnp.float32),
            pltpu.VMEM((T // _NW,), jnp.int32),
            pltpu.VMEM((T // _NW,), jnp.int32),
            pltpu.VMEM((T // _NW,), jnp.int32),
            pltpu.VMEM((T // _NW,), jnp.int32),
            pltpu.VMEM((E,), jnp.int32),
            pltpu.VMEM((CH_A,), jnp.int32),
            pltpu.VMEM((CH_A,), jnp.int32),
            pltpu.SemaphoreType.DMA,
        ],
    )
    xs = dispatch(x, e0, e1, r0, r1, off)

    b1r = b1.reshape(E, 1, H)
    b2r = b2.reshape(E, 1, D)
    grid_spec = pltpu.PrefetchScalarGridSpec(
        num_scalar_prefetch=1,
        grid=(NT,),
        in_specs=[
            pl.BlockSpec((TILE_R, D), lambda i, g: (i, 0)),
            pl.BlockSpec((1, D, H), lambda i, g: (g[i], 0, 0)),
            pl.BlockSpec((1, 1, H), lambda i, g: (g[i], 0, 0)),
            pl.BlockSpec((1, H, D), lambda i, g: (g[i], 0, 0)),
            pl.BlockSpec((1, 1, D), lambda i, g: (g[i], 0, 0)),
        ],
        out_specs=pl.BlockSpec((TILE_R, D), lambda i, g: (i, 0)),
    )
    ys = pl.pallas_call(
        _mlp_body,
        grid_spec=grid_spec,
        out_shape=jax.ShapeDtypeStruct((PADDED, D), jnp.float32),
    )(gids, xs, W1, b1r, W2, b2r)

    CH_B = 16
    combine = pl.kernel(
        functools.partial(_sc_combine_body, T, D, E, CH_B),
        mesh=mesh,
        out_type=jax.ShapeDtypeStruct((T, D), jnp.float32),
        scratch_types=[
            pltpu.VMEM((CH_B, D), jnp.float32),
            pltpu.VMEM((CH_B, D), jnp.float32),
            pltpu.VMEM((CH_B, D), jnp.float32),
            pltpu.VMEM((T // _NW,), jnp.int32),
            pltpu.VMEM((T // _NW,), jnp.int32),
            pltpu.VMEM((T // _NW,), jnp.int32),
            pltpu.VMEM((T // _NW,), jnp.int32),
            pltpu.VMEM((T // _NW,), jnp.float32),
            pltpu.VMEM((T // _NW,), jnp.float32),
            pltpu.VMEM((E,), jnp.int32),
            pltpu.VMEM((_L,), jnp.int32),
            pltpu.VMEM((_L,), jnp.int32),
            pltpu.SemaphoreType.DMA,
        ],
    )
    y = combine(x, ys, e0, e1, r0, r1, w0, w1, off)
    return y


# speedup vs baseline: 1.4336x; 1.0632x over previous
import functools
import math

import jax
import jax.numpy as jnp
from jax import lax
from jax.experimental import pallas as pl
from jax.experimental.pallas import tpu as pltpu
from jax.experimental.pallas import tpu_sc as plsc

TILE_R = 128
_NC = 2
_NS = 16
_L = 16
_NW = _NC * _NS


def _router_body(x_ref, wr_ref, br_ref,
                 e0_ref, e1_ref, r0_ref, r1_ref, w0_ref, w1_ref, cnt_ref,
                 carry_ref):
    t = pl.program_id(0)
    nt = pl.num_programs(0)

    @pl.when(t == 0)
    def _():
        carry_ref[...] = jnp.zeros_like(carry_ref)

    xb = x_ref[...]
    logits = jnp.dot(xb, wr_ref[...], preferred_element_type=jnp.float32)
    logits = logits + br_ref[...]
    m = jnp.max(logits, axis=-1, keepdims=True)
    p = jnp.exp(logits - m)
    p = p / jnp.sum(p, axis=-1, keepdims=True)
    BT, E = p.shape
    idx = jax.lax.broadcasted_iota(jnp.int32, p.shape, 1)
    p0 = jnp.max(p, axis=-1, keepdims=True)
    e0 = jnp.min(jnp.where(p == p0, idx, E), axis=-1, keepdims=True)
    oh0 = (idx == e0)
    pm = jnp.where(oh0, -1.0, p)
    p1 = jnp.max(pm, axis=-1, keepdims=True)
    e1 = jnp.min(jnp.where(pm == p1, idx, E), axis=-1, keepdims=True)
    oh1 = (idx == e1)
    s = jnp.maximum(p0 + p1, 1e-9)

    oh0f = oh0.astype(jnp.float32)
    oh1f = oh1.astype(jnp.float32)
    h = oh0f + oh1f
    ri = jax.lax.broadcasted_iota(jnp.int32, (BT, BT), 0)
    ci = jax.lax.broadcasted_iota(jnp.int32, (BT, BT), 1)
    tril = (ci < ri).astype(jnp.float32)
    C = jnp.dot(tril, h, preferred_element_type=jnp.float32)
    carry = carry_ref[0:1, :]
    Cg = C + carry
    r0 = jnp.sum(oh0f * Cg, axis=-1, keepdims=True)
    r1 = jnp.sum(oh1f * Cg, axis=-1, keepdims=True)

    e0_ref[...] = e0.reshape(1, 1, BT)
    e1_ref[...] = e1.reshape(1, 1, BT)
    r0_ref[...] = r0.astype(jnp.int32).reshape(1, 1, BT)
    r1_ref[...] = r1.astype(jnp.int32).reshape(1, 1, BT)
    w0_ref[...] = (p0 / s).reshape(1, 1, BT)
    w1_ref[...] = (p1 / s).reshape(1, 1, BT)

    new_carry = carry + jnp.sum(h, axis=0, keepdims=True)
    carry_ref[0:1, :] = new_carry

    @pl.when(t == nt - 1)
    def _():
        cnt_ref[...] = new_carry.astype(jnp.int32)


def _mlp_body(gid_ref, xs_ref, w1_ref, b1_ref, w2_ref, b2_ref, out_ref):
    xb = xs_ref[...]
    h = jnp.dot(xb, w1_ref[0], preferred_element_type=jnp.float32) + b1_ref[0]
    h = 0.5 * h * (1.0 + jax.lax.erf(h * (1.0 / math.sqrt(2.0))))
    out_ref[...] = jnp.dot(h, w2_ref[0],
                           preferred_element_type=jnp.float32) + b2_ref[0]


def _vgather(vec, idx):
    dnums = lax.GatherDimensionNumbers(
        offset_dims=(), collapsed_slice_dims=(0,), start_index_map=(0,))
    return lax.gather(vec, idx[:, None], dnums, (1,),
                      mode=lax.GatherScatterMode.PROMISE_IN_BOUNDS)

def _sc_dispatch_body(T, D, E, CH,
                      x_hbm, e0_hbm, e1_hbm, r0_hbm, r1_hbm, off_hbm, xs_hbm,
                      xv, e0v, e1v, r0v, r1v, offv, idx0, idx1, sem):
    tpw = T // _NW
    wid = lax.axis_index("s") * _NC + lax.axis_index("c")
    t0 = pl.multiple_of(wid * tpw, tpw)
    pltpu.sync_copy(e0_hbm.at[pl.ds(t0, tpw)], e0v)
    pltpu.sync_copy(e1_hbm.at[pl.ds(t0, tpw)], e1v)
    pltpu.sync_copy(r0_hbm.at[pl.ds(t0, tpw)], r0v)
    pltpu.sync_copy(r1_hbm.at[pl.ds(t0, tpw)], r1v)
    pltpu.sync_copy(off_hbm, offv)
    off_vec = offv[...]

    def chunk(c, carry):
        cb = pl.multiple_of(c * CH, CH)
        pltpu.sync_copy(x_hbm.at[pl.ds(t0 + cb, CH)], xv)
        for g in range(CH // _L):
            o = pl.multiple_of(cb + g * _L, _L)
            pos0 = _vgather(off_vec, e0v[pl.ds(o, _L)]) + r0v[pl.ds(o, _L)]
            idx0[pl.ds(g * _L, _L)] = pos0
            pos1 = _vgather(off_vec, e1v[pl.ds(o, _L)]) + r1v[pl.ds(o, _L)]
            idx1[pl.ds(g * _L, _L)] = pos1
        cp0 = pltpu.async_copy(xv, xs_hbm.at[idx0], sem)
        cp1 = pltpu.async_copy(xv, xs_hbm.at[idx1], sem)
        cp0.wait()
        cp1.wait()
        return carry

    lax.fori_loop(0, tpw // CH, chunk, 0)


def _sc_combine_body(T, D, E, CH,
                     x_hbm, ys_hbm, e0_hbm, e1_hbm, r0_hbm, r1_hbm,
                     w0_hbm, w1_hbm, off_hbm, y_hbm,
                     xv0, xv1, av0, av1, bv0, bv1,
                     e0v, e1v, r0v, r1v, w0v, w1v, offv,
                     idx00, idx10, idx01, idx11, sem0, sem1):
    tpw = T // _NW
    nch = tpw // CH
    wid = lax.axis_index("s") * _NC + lax.axis_index("c")
    t0 = pl.multiple_of(wid * tpw, tpw)
    pltpu.sync_copy(e0_hbm.at[pl.ds(t0, tpw)], e0v)
    pltpu.sync_copy(e1_hbm.at[pl.ds(t0, tpw)], e1v)
    pltpu.sync_copy(r0_hbm.at[pl.ds(t0, tpw)], r0v)
    pltpu.sync_copy(r1_hbm.at[pl.ds(t0, tpw)], r1v)
    pltpu.sync_copy(w0_hbm.at[pl.ds(t0, tpw)], w0v)
    pltpu.sync_copy(w1_hbm.at[pl.ds(t0, tpw)], w1v)
    pltpu.sync_copy(off_hbm, offv)
    off_vec = offv[...]

    bufs = ((xv0, av0, bv0, idx00, idx10, sem0),
            (xv1, av1, bv1, idx01, idx11, sem1))

    def fire(c, b):
        xv, av, bv, idx0, idx1, sem = bufs[b]
        cb = pl.multiple_of(c * CH, CH)
        pos0 = _vgather(off_vec, e0v[pl.ds(cb, _L)]) + r0v[pl.ds(cb, _L)]
        idx0[...] = pos0
        pos1 = _vgather(off_vec, e1v[pl.ds(cb, _L)]) + r1v[pl.ds(cb, _L)]
        idx1[...] = pos1
        pltpu.async_copy(ys_hbm.at[idx0], av, sem)
        pltpu.async_copy(ys_hbm.at[idx1], bv, sem)
        pltpu.async_copy(x_hbm.at[pl.ds(t0 + cb, CH)], xv, sem)

    def drain(b):
        xv, av, bv, idx0, idx1, sem = bufs[b]
        pltpu.make_async_copy(ys_hbm.at[idx0], av, sem).wait()
        pltpu.make_async_copy(ys_hbm.at[idx1], bv, sem).wait()
        pltpu.make_async_copy(x_hbm.at[pl.ds(t0, CH)], xv, sem).wait()

    def compute(c, b):
        xv, av, bv, idx0, idx1, sem = bufs[b]
        cb = pl.multiple_of(c * CH, CH)
        w0c = w0v[pl.ds(cb, _L)]
        w1c = w1v[pl.ds(cb, _L)]
        for j in range(CH):
            jv = jnp.full((_L,), j, jnp.int32)
            wa = _vgather(w0c, jv)
            wb = _vgather(w1c, jv)

            UN = 8

            def vg(v, cc, j=j, wa=wa, wb=wb, xv=xv, av=av, bv=bv):
                for u in range(UN):
                    sl = pl.ds(pl.multiple_of(v * (_L * UN), _L) + u * _L, _L)
                    xv[j, sl] = xv[j, sl] + wa * av[j, sl] + wb * bv[j, sl]
                return cc

            lax.fori_loop(0, D // (_L * UN), vg, 0)
        pltpu.sync_copy(xv, y_hbm.at[pl.ds(t0 + cb, CH)])

    fire(0, 0)

    def outer(g, carry):
        c0 = pl.multiple_of(g * 2, 2)
        fire(c0 + 1, 1)
        drain(0)
        compute(c0, 0)

        @pl.when(g < nch // 2 - 1)
        def _():
            fire(c0 + 2, 0)

        drain(1)
        compute(c0 + 1, 1)
        return carry

    lax.fori_loop(0, nch // 2, outer, 0)


def kernel(x, Wr, br, W1, b1, W2, b2):
    T, D = x.shape
    E = Wr.shape[1]
    H = W1.shape[2]
    BT_R = 1024
    nblk = T // BT_R

    outs = pl.pallas_call(
        _router_body,
        grid=(nblk,),
        in_specs=[
            pl.BlockSpec((BT_R, D), lambda t: (t, 0)),
            pl.BlockSpec((D, E), lambda t: (0, 0)),
            pl.BlockSpec((E,), lambda t: (0,)),
        ],
        out_specs=[
            pl.BlockSpec((1, 1, BT_R), lambda t: (t, 0, 0)),
            pl.BlockSpec((1, 1, BT_R), lambda t: (t, 0, 0)),
            pl.BlockSpec((1, 1, BT_R), lambda t: (t, 0, 0)),
            pl.BlockSpec((1, 1, BT_R), lambda t: (t, 0, 0)),
            pl.BlockSpec((1, 1, BT_R), lambda t: (t, 0, 0)),
            pl.BlockSpec((1, 1, BT_R), lambda t: (t, 0, 0)),
            pl.BlockSpec((1, E), lambda t: (0, 0)),
        ],
        out_shape=[
            jax.ShapeDtypeStruct((nblk, 1, BT_R), jnp.int32),
            jax.ShapeDtypeStruct((nblk, 1, BT_R), jnp.int32),
            jax.ShapeDtypeStruct((nblk, 1, BT_R), jnp.int32),
            jax.ShapeDtypeStruct((nblk, 1, BT_R), jnp.int32),
            jax.ShapeDtypeStruct((nblk, 1, BT_R), jnp.float32),
            jax.ShapeDtypeStruct((nblk, 1, BT_R), jnp.float32),
            jax.ShapeDtypeStruct((1, E), jnp.int32),
        ],
        scratch_shapes=[pltpu.VMEM((8, E), jnp.float32)],
    )(x, Wr, br)
    e0, e1, r0, r1, w0, w1, counts = outs
    e0 = e0.reshape(T)
    e1 = e1.reshape(T)
    r0 = r0.reshape(T)
    r1 = r1.reshape(T)
    w0 = w0.reshape(T)
    w1 = w1.reshape(T)
    counts = counts.reshape(E)

    padded = ((counts + TILE_R - 1) // TILE_R) * TILE_R
    ends = jnp.cumsum(padded)
    off = (ends - padded).astype(jnp.int32)
    PADDED = T * 2 + E * TILE_R
    NT = PADDED // TILE_R
    tile_starts = jnp.arange(NT, dtype=jnp.int32) * TILE_R
    gids = jnp.clip(jnp.searchsorted(ends, tile_starts, side="right"),
                    0, E - 1).astype(jnp.int32)

    mesh = plsc.VectorSubcoreMesh(core_axis_name="c", subcore_axis_name="s")
    CH_A = 32
    dispatch = pl.kernel(
        functools.partial(_sc_dispatch_body, T, D, E, CH_A),
        mesh=mesh,
        out_type=jax.ShapeDtypeStruct((PADDED, D), jnp.float32),
        scratch_types=[
            pltpu.VMEM((CH_A, D), jnp.float32),
            pltpu.VMEM((T // _NW,), jnp.int32),
            pltpu.VMEM((T // _NW,), jnp.int32),
            pltpu.VMEM((T // _NW,), jnp.int32),
            pltpu.VMEM((T // _NW,), jnp.int32),
            pltpu.VMEM((E,), jnp.int32),
            pltpu.VMEM((CH_A,), jnp.int32),
            pltpu.VMEM((CH_A,), jnp.int32),
            pltpu.SemaphoreType.DMA,
        ],
    )
    xs = dispatch(x, e0, e1, r0, r1, off)

    b1r = b1.reshape(E, 1, H)
    b2r = b2.reshape(E, 1, D)
    grid_spec = pltpu.PrefetchScalarGridSpec(
        num_scalar_prefetch=1,
        grid=(NT,),
        in_specs=[
            pl.BlockSpec((TILE_R, D), lambda i, g: (i, 0)),
            pl.BlockSpec((1, D, H), lambda i, g: (g[i], 0, 0)),
            pl.BlockSpec((1, 1, H), lambda i, g: (g[i], 0, 0)),
            pl.BlockSpec((1, H, D), lambda i, g: (g[i], 0, 0)),
            pl.BlockSpec((1, 1, D), lambda i, g: (g[i], 0, 0)),
        ],
        out_specs=pl.BlockSpec((TILE_R, D), lambda i, g: (i, 0)),
    )
    ys = pl.pallas_call(
        _mlp_body,
        grid_spec=grid_spec,
        out_shape=jax.ShapeDtypeStruct((PADDED, D), jnp.float32),
    )(gids, xs, W1, b1r, W2, b2r)

    CH_B = 16
    combine = pl.kernel(
        functools.partial(_sc_combine_body, T, D, E, CH_B),
        mesh=mesh,
        out_type=jax.ShapeDtypeStruct((T, D), jnp.float32),
        scratch_types=[
            pltpu.VMEM((CH_B, D), jnp.float32),
            pltpu.VMEM((CH_B, D), jnp.float32),
            pltpu.VMEM((CH_B, D), jnp.float32),
            pltpu.VMEM((CH_B, D), jnp.float32),
            pltpu.VMEM((CH_B, D), jnp.float32),
            pltpu.VMEM((CH_B, D), jnp.float32),
            pltpu.VMEM((T // _NW,), jnp.int32),
            pltpu.VMEM((T // _NW,), jnp.int32),
            pltpu.VMEM((T // _NW,), jnp.int32),
            pltpu.VMEM((T // _NW,), jnp.int32),
            pltpu.VMEM((T // _NW,), jnp.float32),
            pltpu.VMEM((T // _NW,), jnp.float32),
            pltpu.VMEM((E,), jnp.int32),
            pltpu.VMEM((_L,), jnp.int32),
            pltpu.VMEM((_L,), jnp.int32),
            pltpu.VMEM((_L,), jnp.int32),
            pltpu.VMEM((_L,), jnp.int32),
            pltpu.SemaphoreType.DMA,
            pltpu.SemaphoreType.DMA,
        ],
    )
    y = combine(x, ys, e0, e1, r0, r1, w0, w1, off)
    return y


# speedup vs baseline: 1.4401x; 1.0046x over previous
import functools
import math

import jax
import jax.numpy as jnp
from jax import lax
from jax.experimental import pallas as pl
from jax.experimental.pallas import tpu as pltpu
from jax.experimental.pallas import tpu_sc as plsc

TILE_R = 128
_NC = 2
_NS = 16
_L = 16
_NW = _NC * _NS


def _router_body(x_ref, wr_ref, br_ref,
                 e0_ref, e1_ref, r0_ref, r1_ref, w0_ref, w1_ref, cnt_ref,
                 carry_ref):
    t = pl.program_id(0)
    nt = pl.num_programs(0)

    @pl.when(t == 0)
    def _():
        carry_ref[...] = jnp.zeros_like(carry_ref)

    xb = x_ref[...]
    logits = jnp.dot(xb, wr_ref[...], preferred_element_type=jnp.float32)
    logits = logits + br_ref[...]
    m = jnp.max(logits, axis=-1, keepdims=True)
    p = jnp.exp(logits - m)
    p = p / jnp.sum(p, axis=-1, keepdims=True)
    BT, E = p.shape
    idx = jax.lax.broadcasted_iota(jnp.int32, p.shape, 1)
    p0 = jnp.max(p, axis=-1, keepdims=True)
    e0 = jnp.min(jnp.where(p == p0, idx, E), axis=-1, keepdims=True)
    oh0 = (idx == e0)
    pm = jnp.where(oh0, -1.0, p)
    p1 = jnp.max(pm, axis=-1, keepdims=True)
    e1 = jnp.min(jnp.where(pm == p1, idx, E), axis=-1, keepdims=True)
    oh1 = (idx == e1)
    s = jnp.maximum(p0 + p1, 1e-9)

    oh0f = oh0.astype(jnp.float32)
    oh1f = oh1.astype(jnp.float32)
    h = oh0f + oh1f
    ri = jax.lax.broadcasted_iota(jnp.int32, (BT, BT), 0)
    ci = jax.lax.broadcasted_iota(jnp.int32, (BT, BT), 1)
    tril = (ci < ri).astype(jnp.float32)
    C = jnp.dot(tril, h, preferred_element_type=jnp.float32)
    carry = carry_ref[0:1, :]
    Cg = C + carry
    r0 = jnp.sum(oh0f * Cg, axis=-1, keepdims=True)
    r1 = jnp.sum(oh1f * Cg, axis=-1, keepdims=True)

    e0_ref[...] = e0.reshape(1, 1, BT)
    e1_ref[...] = e1.reshape(1, 1, BT)
    r0_ref[...] = r0.astype(jnp.int32).reshape(1, 1, BT)
    r1_ref[...] = r1.astype(jnp.int32).reshape(1, 1, BT)
    w0_ref[...] = (p0 / s).reshape(1, 1, BT)
    w1_ref[...] = (p1 / s).reshape(1, 1, BT)

    new_carry = carry + jnp.sum(h, axis=0, keepdims=True)
    carry_ref[0:1, :] = new_carry

    @pl.when(t == nt - 1)
    def _():
        cnt_ref[...] = new_carry.astype(jnp.int32)


def _mlp_body(gid_ref, xs_ref, w1_ref, b1_ref, w2_ref, b2_ref, out_ref):
    @pl.when(pl.program_id(0) < gid_ref[pl.num_programs(0)])
    def _():
        xb = xs_ref[...]
        h = jnp.dot(xb, w1_ref[0], preferred_element_type=jnp.float32) + b1_ref[0]
        h = 0.5 * h * (1.0 + jax.lax.erf(h * (1.0 / math.sqrt(2.0))))
        out_ref[...] = jnp.dot(h, w2_ref[0],
                               preferred_element_type=jnp.float32) + b2_ref[0]


def _vgather(vec, idx):
    dnums = lax.GatherDimensionNumbers(
        offset_dims=(), collapsed_slice_dims=(0,), start_index_map=(0,))
    return lax.gather(vec, idx[:, None], dnums, (1,),
                      mode=lax.GatherScatterMode.PROMISE_IN_BOUNDS)

def _sc_dispatch_body(T, D, E, CH,
                      x_hbm, e0_hbm, e1_hbm, r0_hbm, r1_hbm, off_hbm, xs_hbm,
                      xv0, xv1, e0v, e1v, r0v, r1v, offv,
                      idx00, idx10, idx01, idx11,
                      seml0, seml1, sems0, sems1):
    tpw = T // _NW
    nch = tpw // CH
    wid = lax.axis_index("s") * _NC + lax.axis_index("c")
    t0 = pl.multiple_of(wid * tpw, tpw)
    pltpu.sync_copy(e0_hbm.at[pl.ds(t0, tpw)], e0v)
    pltpu.sync_copy(e1_hbm.at[pl.ds(t0, tpw)], e1v)
    pltpu.sync_copy(r0_hbm.at[pl.ds(t0, tpw)], r0v)
    pltpu.sync_copy(r1_hbm.at[pl.ds(t0, tpw)], r1v)
    pltpu.sync_copy(off_hbm, offv)
    off_vec = offv[...]

    bufs = ((xv0, idx00, idx10, seml0, sems0),
            (xv1, idx01, idx11, seml1, sems1))

    def fire_load(c, b):
        xv, idx0, idx1, seml, sems = bufs[b]
        cb = pl.multiple_of(c * CH, CH)
        pltpu.async_copy(x_hbm.at[pl.ds(t0 + cb, CH)], xv, seml)

    def scatter(c, b):
        xv, idx0, idx1, seml, sems = bufs[b]
        cb = pl.multiple_of(c * CH, CH)
        pltpu.make_async_copy(x_hbm.at[pl.ds(t0, CH)], xv, seml).wait()
        for g in range(CH // _L):
            o = pl.multiple_of(cb + g * _L, _L)
            pos0 = _vgather(off_vec, e0v[pl.ds(o, _L)]) + r0v[pl.ds(o, _L)]
            idx0[pl.ds(g * _L, _L)] = pos0
            pos1 = _vgather(off_vec, e1v[pl.ds(o, _L)]) + r1v[pl.ds(o, _L)]
            idx1[pl.ds(g * _L, _L)] = pos1
        pltpu.async_copy(xv, xs_hbm.at[idx0], sems)
        pltpu.async_copy(xv, xs_hbm.at[idx1], sems)

    def drain_scatter(b):
        xv, idx0, idx1, seml, sems = bufs[b]
        pltpu.make_async_copy(xv, xs_hbm.at[idx0], sems).wait()
        pltpu.make_async_copy(xv, xs_hbm.at[idx1], sems).wait()

    fire_load(0, 0)
    fire_load(1, 1)

    def outer(g, carry):
        c0 = pl.multiple_of(g * 2, 2)
        scatter(c0, 0)
        scatter(c0 + 1, 1)
        drain_scatter(0)

        @pl.when(g < nch // 2 - 1)
        def _():
            fire_load(c0 + 2, 0)

        drain_scatter(1)

        @pl.when(g < nch // 2 - 1)
        def _():
            fire_load(c0 + 3, 1)

        return carry

    lax.fori_loop(0, nch // 2, outer, 0)


def _sc_combine_body(T, D, E, CH,
                     x_hbm, ys_hbm, e0_hbm, e1_hbm, r0_hbm, r1_hbm,
                     w0_hbm, w1_hbm, off_hbm, y_hbm,
                     xv0, xv1, av0, av1, bv0, bv1,
                     e0v, e1v, r0v, r1v, w0v, w1v, offv,
                     idx00, idx10, idx01, idx11, sem0, sem1):
    tpw = T // _NW
    nch = tpw // CH
    wid = lax.axis_index("s") * _NC + lax.axis_index("c")
    t0 = pl.multiple_of(wid * tpw, tpw)
    pltpu.sync_copy(e0_hbm.at[pl.ds(t0, tpw)], e0v)
    pltpu.sync_copy(e1_hbm.at[pl.ds(t0, tpw)], e1v)
    pltpu.sync_copy(r0_hbm.at[pl.ds(t0, tpw)], r0v)
    pltpu.sync_copy(r1_hbm.at[pl.ds(t0, tpw)], r1v)
    pltpu.sync_copy(w0_hbm.at[pl.ds(t0, tpw)], w0v)
    pltpu.sync_copy(w1_hbm.at[pl.ds(t0, tpw)], w1v)
    pltpu.sync_copy(off_hbm, offv)
    off_vec = offv[...]

    bufs = ((xv0, av0, bv0, idx00, idx10, sem0),
            (xv1, av1, bv1, idx01, idx11, sem1))

    def fire(c, b):
        xv, av, bv, idx0, idx1, sem = bufs[b]
        cb = pl.multiple_of(c * CH, CH)
        pos0 = _vgather(off_vec, e0v[pl.ds(cb, _L)]) + r0v[pl.ds(cb, _L)]
        idx0[...] = pos0
        pos1 = _vgather(off_vec, e1v[pl.ds(cb, _L)]) + r1v[pl.ds(cb, _L)]
        idx1[...] = pos1
        pltpu.async_copy(ys_hbm.at[idx0], av, sem)
        pltpu.async_copy(ys_hbm.at[idx1], bv, sem)
        pltpu.async_copy(x_hbm.at[pl.ds(t0 + cb, CH)], xv, sem)

    def drain(b):
        xv, av, bv, idx0, idx1, sem = bufs[b]
        pltpu.make_async_copy(ys_hbm.at[idx0], av, sem).wait()
        pltpu.make_async_copy(ys_hbm.at[idx1], bv, sem).wait()
        pltpu.make_async_copy(x_hbm.at[pl.ds(t0, CH)], xv, sem).wait()

    def compute(c, b):
        xv, av, bv, idx0, idx1, sem = bufs[b]
        cb = pl.multiple_of(c * CH, CH)
        w0c = w0v[pl.ds(cb, _L)]
        w1c = w1v[pl.ds(cb, _L)]
        for j in range(CH):
            jv = jnp.full((_L,), j, jnp.int32)
            wa = _vgather(w0c, jv)
            wb = _vgather(w1c, jv)

            UN = 8

            def vg(v, cc, j=j, wa=wa, wb=wb, xv=xv, av=av, bv=bv):
                for u in range(UN):
                    sl = pl.ds(pl.multiple_of(v * (_L * UN), _L) + u * _L, _L)
                    xv[j, sl] = xv[j, sl] + wa * av[j, sl] + wb * bv[j, sl]
                return cc

            lax.fori_loop(0, D // (_L * UN), vg, 0)
        pltpu.sync_copy(xv, y_hbm.at[pl.ds(t0 + cb, CH)])

    fire(0, 0)

    def outer(g, carry):
        c0 = pl.multiple_of(g * 2, 2)
        fire(c0 + 1, 1)
        drain(0)
        compute(c0, 0)

        @pl.when(g < nch // 2 - 1)
        def _():
            fire(c0 + 2, 0)

        drain(1)
        compute(c0 + 1, 1)
        return carry

    lax.fori_loop(0, nch // 2, outer, 0)


def kernel(x, Wr, br, W1, b1, W2, b2):
    T, D = x.shape
    E = Wr.shape[1]
    H = W1.shape[2]
    BT_R = 1024
    nblk = T // BT_R

    outs = pl.pallas_call(
        _router_body,
        grid=(nblk,),
        in_specs=[
            pl.BlockSpec((BT_R, D), lambda t: (t, 0)),
            pl.BlockSpec((D, E), lambda t: (0, 0)),
            pl.BlockSpec((E,), lambda t: (0,)),
        ],
        out_specs=[
            pl.BlockSpec((1, 1, BT_R), lambda t: (t, 0, 0)),
            pl.BlockSpec((1, 1, BT_R), lambda t: (t, 0, 0)),
            pl.BlockSpec((1, 1, BT_R), lambda t: (t, 0, 0)),
            pl.BlockSpec((1, 1, BT_R), lambda t: (t, 0, 0)),
            pl.BlockSpec((1, 1, BT_R), lambda t: (t, 0, 0)),
            pl.BlockSpec((1, 1, BT_R), lambda t: (t, 0, 0)),
            pl.BlockSpec((1, E), lambda t: (0, 0)),
        ],
        out_shape=[
            jax.ShapeDtypeStruct((nblk, 1, BT_R), jnp.int32),
            jax.ShapeDtypeStruct((nblk, 1, BT_R), jnp.int32),
            jax.ShapeDtypeStruct((nblk, 1, BT_R), jnp.int32),
            jax.ShapeDtypeStruct((nblk, 1, BT_R), jnp.int32),
            jax.ShapeDtypeStruct((nblk, 1, BT_R), jnp.float32),
            jax.ShapeDtypeStruct((nblk, 1, BT_R), jnp.float32),
            jax.ShapeDtypeStruct((1, E), jnp.int32),
        ],
        scratch_shapes=[pltpu.VMEM((8, E), jnp.float32)],
    )(x, Wr, br)
    e0, e1, r0, r1, w0, w1, counts = outs
    e0 = e0.reshape(T)
    e1 = e1.reshape(T)
    r0 = r0.reshape(T)
    r1 = r1.reshape(T)
    w0 = w0.reshape(T)
    w1 = w1.reshape(T)
    counts = counts.reshape(E)

    padded = ((counts + TILE_R - 1) // TILE_R) * TILE_R
    ends = jnp.cumsum(padded)
    off = (ends - padded).astype(jnp.int32)
    PADDED = T * 2 + E * TILE_R
    NT = PADDED // TILE_R
    tile_starts = jnp.arange(NT, dtype=jnp.int32) * TILE_R
    gids = jnp.clip(jnp.searchsorted(ends, tile_starts, side="right"),
                    0, E - 1).astype(jnp.int32)
    n_used = (ends[E - 1] // TILE_R).astype(jnp.int32)
    gids = jnp.concatenate([gids, n_used[None]])

    mesh = plsc.VectorSubcoreMesh(core_axis_name="c", subcore_axis_name="s")
    CH_A = 32
    dispatch = pl.kernel(
        functools.partial(_sc_dispatch_body, T, D, E, CH_A),
        mesh=mesh,
        out_type=jax.ShapeDtypeStruct((PADDED, D), jnp.float32),
        scratch_types=[
            pltpu.VMEM((CH_A, D), jnp.float32),
            pltpu.VMEM((CH_A, D), jnp.float32),
            pltpu.VMEM((T // _NW,), jnp.int32),
            pltpu.VMEM((T // _NW,), jnp.int32),
            pltpu.VMEM((T // _NW,), jnp.int32),
            pltpu.VMEM((T // _NW,), jnp.int32),
            pltpu.VMEM((E,), jnp.int32),
            pltpu.VMEM((CH_A,), jnp.int32),
            pltpu.VMEM((CH_A,), jnp.int32),
            pltpu.VMEM((CH_A,), jnp.int32),
            pltpu.VMEM((CH_A,), jnp.int32),
            pltpu.SemaphoreType.DMA,
            pltpu.SemaphoreType.DMA,
            pltpu.SemaphoreType.DMA,
            pltpu.SemaphoreType.DMA,
        ],
    )
    xs = dispatch(x, e0, e1, r0, r1, off)

    b1r = b1.reshape(E, 1, H)
    b2r = b2.reshape(E, 1, D)
    grid_spec = pltpu.PrefetchScalarGridSpec(
        num_scalar_prefetch=1,
        grid=(NT,),
        in_specs=[
            pl.BlockSpec((TILE_R, D), lambda i, g: (i, 0)),
            pl.BlockSpec((1, D, H), lambda i, g: (g[i], 0, 0)),
            pl.BlockSpec((1, 1, H), lambda i, g: (g[i], 0, 0)),
            pl.BlockSpec((1, H, D), lambda i, g: (g[i], 0, 0)),
            pl.BlockSpec((1, 1, D), lambda i, g: (g[i], 0, 0)),
        ],
        out_specs=pl.BlockSpec((TILE_R, D), lambda i, g: (i, 0)),
    )
    ys = pl.pallas_call(
        _mlp_body,
        grid_spec=grid_spec,
        out_shape=jax.ShapeDtypeStruct((PADDED, D), jnp.float32),
    )(gids, xs, W1, b1r, W2, b2r)

    CH_B = 16
    combine = pl.kernel(
        functools.partial(_sc_combine_body, T, D, E, CH_B),
        mesh=mesh,
        out_type=jax.ShapeDtypeStruct((T, D), jnp.float32),
        scratch_types=[
            pltpu.VMEM((CH_B, D), jnp.float32),
            pltpu.VMEM((CH_B, D), jnp.float32),
            pltpu.VMEM((CH_B, D), jnp.float32),
            pltpu.VMEM((CH_B, D), jnp.float32),
            pltpu.VMEM((CH_B, D), jnp.float32),
            pltpu.VMEM((CH_B, D), jnp.float32),
            pltpu.VMEM((T // _NW,), jnp.int32),
            pltpu.VMEM((T // _NW,), jnp.int32),
            pltpu.VMEM((T // _NW,), jnp.int32),
            pltpu.VMEM((T // _NW,), jnp.int32),
            pltpu.VMEM((T // _NW,), jnp.float32),
            pltpu.VMEM((T // _NW,), jnp.float32),
            pltpu.VMEM((E,), jnp.int32),
            pltpu.VMEM((_L,), jnp.int32),
            pltpu.VMEM((_L,), jnp.int32),
            pltpu.VMEM((_L,), jnp.int32),
            pltpu.VMEM((_L,), jnp.int32),
            pltpu.SemaphoreType.DMA,
            pltpu.SemaphoreType.DMA,
        ],
    )
    y = combine(x, ys, e0, e1, r0, r1, w0, w1, off)
    return y


# speedup vs baseline: 1.5090x; 1.0478x over previous
import functools
import math

import jax
import jax.numpy as jnp
from jax import lax
from jax.experimental import pallas as pl
from jax.experimental.pallas import tpu as pltpu
from jax.experimental.pallas import tpu_sc as plsc

TILE_R = 256
_NC = 2
_NS = 16
_L = 16
_NW = _NC * _NS


def _router_body(x_ref, wr_ref, br_ref,
                 e0_ref, e1_ref, r0_ref, r1_ref, w0_ref, w1_ref, cnt_ref,
                 carry_ref):
    t = pl.program_id(0)
    nt = pl.num_programs(0)

    @pl.when(t == 0)
    def _():
        carry_ref[...] = jnp.zeros_like(carry_ref)

    xb = x_ref[...]
    logits = jnp.dot(xb, wr_ref[...], preferred_element_type=jnp.float32)
    logits = logits + br_ref[...]
    m = jnp.max(logits, axis=-1, keepdims=True)
    p = jnp.exp(logits - m)
    p = p / jnp.sum(p, axis=-1, keepdims=True)
    BT, E = p.shape
    idx = jax.lax.broadcasted_iota(jnp.int32, p.shape, 1)
    p0 = jnp.max(p, axis=-1, keepdims=True)
    e0 = jnp.min(jnp.where(p == p0, idx, E), axis=-1, keepdims=True)
    oh0 = (idx == e0)
    pm = jnp.where(oh0, -1.0, p)
    p1 = jnp.max(pm, axis=-1, keepdims=True)
    e1 = jnp.min(jnp.where(pm == p1, idx, E), axis=-1, keepdims=True)
    oh1 = (idx == e1)
    s = jnp.maximum(p0 + p1, 1e-9)

    oh0f = oh0.astype(jnp.float32)
    oh1f = oh1.astype(jnp.float32)
    h = oh0f + oh1f
    ri = jax.lax.broadcasted_iota(jnp.int32, (BT, BT), 0)
    ci = jax.lax.broadcasted_iota(jnp.int32, (BT, BT), 1)
    tril = (ci < ri).astype(jnp.float32)
    C = jnp.dot(tril, h, preferred_element_type=jnp.float32)
    carry = carry_ref[0:1, :]
    Cg = C + carry
    r0 = jnp.sum(oh0f * Cg, axis=-1, keepdims=True)
    r1 = jnp.sum(oh1f * Cg, axis=-1, keepdims=True)

    e0_ref[...] = e0.reshape(1, 1, BT)
    e1_ref[...] = e1.reshape(1, 1, BT)
    r0_ref[...] = r0.astype(jnp.int32).reshape(1, 1, BT)
    r1_ref[...] = r1.astype(jnp.int32).reshape(1, 1, BT)
    w0_ref[...] = (p0 / s).reshape(1, 1, BT)
    w1_ref[...] = (p1 / s).reshape(1, 1, BT)

    new_carry = carry + jnp.sum(h, axis=0, keepdims=True)
    carry_ref[0:1, :] = new_carry

    @pl.when(t == nt - 1)
    def _():
        cnt_ref[...] = new_carry.astype(jnp.int32)


def _mlp_body(gid_ref, xs_ref, w1_ref, b1_ref, w2_ref, b2_ref, out_ref):
    @pl.when(pl.program_id(0) < gid_ref[pl.num_programs(0)])
    def _():
        xb = xs_ref[...]
        h = jnp.dot(xb, w1_ref[0], preferred_element_type=jnp.float32) + b1_ref[0]
        h = 0.5 * h * (1.0 + jax.lax.erf(h * (1.0 / math.sqrt(2.0))))
        out_ref[...] = jnp.dot(h, w2_ref[0],
                               preferred_element_type=jnp.float32) + b2_ref[0]


def _vgather(vec, idx):
    dnums = lax.GatherDimensionNumbers(
        offset_dims=(), collapsed_slice_dims=(0,), start_index_map=(0,))
    return lax.gather(vec, idx[:, None], dnums, (1,),
                      mode=lax.GatherScatterMode.PROMISE_IN_BOUNDS)

def _sc_dispatch_body(T, D, E, CH,
                      x_hbm, e0_hbm, e1_hbm, r0_hbm, r1_hbm, off_hbm, xs_hbm,
                      xv0, xv1, e0v, e1v, r0v, r1v, offv,
                      idx00, idx10, idx01, idx11,
                      seml0, seml1, sems0, sems1):
    tpw = T // _NW
    nch = tpw // CH
    wid = lax.axis_index("s") * _NC + lax.axis_index("c")
    t0 = pl.multiple_of(wid * tpw, tpw)
    pltpu.sync_copy(e0_hbm.at[pl.ds(t0, tpw)], e0v)
    pltpu.sync_copy(e1_hbm.at[pl.ds(t0, tpw)], e1v)
    pltpu.sync_copy(r0_hbm.at[pl.ds(t0, tpw)], r0v)
    pltpu.sync_copy(r1_hbm.at[pl.ds(t0, tpw)], r1v)
    pltpu.sync_copy(off_hbm, offv)
    off_vec = offv[...]

    bufs = ((xv0, idx00, idx10, seml0, sems0),
            (xv1, idx01, idx11, seml1, sems1))

    def fire_load(c, b):
        xv, idx0, idx1, seml, sems = bufs[b]
        cb = pl.multiple_of(c * CH, CH)
        pltpu.async_copy(x_hbm.at[pl.ds(t0 + cb, CH)], xv, seml)

    def scatter(c, b):
        xv, idx0, idx1, seml, sems = bufs[b]
        cb = pl.multiple_of(c * CH, CH)
        pltpu.make_async_copy(x_hbm.at[pl.ds(t0, CH)], xv, seml).wait()
        for g in range(CH // _L):
            o = pl.multiple_of(cb + g * _L, _L)
            pos0 = _vgather(off_vec, e0v[pl.ds(o, _L)]) + r0v[pl.ds(o, _L)]
            idx0[pl.ds(g * _L, _L)] = pos0
            pos1 = _vgather(off_vec, e1v[pl.ds(o, _L)]) + r1v[pl.ds(o, _L)]
            idx1[pl.ds(g * _L, _L)] = pos1
        pltpu.async_copy(xv, xs_hbm.at[idx0], sems)
        pltpu.async_copy(xv, xs_hbm.at[idx1], sems)

    def drain_scatter(b):
        xv, idx0, idx1, seml, sems = bufs[b]
        pltpu.make_async_copy(xv, xs_hbm.at[idx0], sems).wait()
        pltpu.make_async_copy(xv, xs_hbm.at[idx1], sems).wait()

    fire_load(0, 0)
    fire_load(1, 1)

    def outer(g, carry):
        c0 = pl.multiple_of(g * 2, 2)
        scatter(c0, 0)
        scatter(c0 + 1, 1)
        drain_scatter(0)

        @pl.when(g < nch // 2 - 1)
        def _():
            fire_load(c0 + 2, 0)

        drain_scatter(1)

        @pl.when(g < nch // 2 - 1)
        def _():
            fire_load(c0 + 3, 1)

        return carry

    lax.fori_loop(0, nch // 2, outer, 0)


def _sc_combine_body(T, D, E, CH,
                     x_hbm, ys_hbm, e0_hbm, e1_hbm, r0_hbm, r1_hbm,
                     w0_hbm, w1_hbm, off_hbm, y_hbm,
                     xv0, xv1, av0, av1, bv0, bv1,
                     e0v, e1v, r0v, r1v, w0v, w1v, offv,
                     idx00, idx10, idx01, idx11, sem0, sem1):
    tpw = T // _NW
    nch = tpw // CH
    wid = lax.axis_index("s") * _NC + lax.axis_index("c")
    t0 = pl.multiple_of(wid * tpw, tpw)
    pltpu.sync_copy(e0_hbm.at[pl.ds(t0, tpw)], e0v)
    pltpu.sync_copy(e1_hbm.at[pl.ds(t0, tpw)], e1v)
    pltpu.sync_copy(r0_hbm.at[pl.ds(t0, tpw)], r0v)
    pltpu.sync_copy(r1_hbm.at[pl.ds(t0, tpw)], r1v)
    pltpu.sync_copy(w0_hbm.at[pl.ds(t0, tpw)], w0v)
    pltpu.sync_copy(w1_hbm.at[pl.ds(t0, tpw)], w1v)
    pltpu.sync_copy(off_hbm, offv)
    off_vec = offv[...]

    bufs = ((xv0, av0, bv0, idx00, idx10, sem0),
            (xv1, av1, bv1, idx01, idx11, sem1))

    def fire(c, b):
        xv, av, bv, idx0, idx1, sem = bufs[b]
        cb = pl.multiple_of(c * CH, CH)
        pos0 = _vgather(off_vec, e0v[pl.ds(cb, _L)]) + r0v[pl.ds(cb, _L)]
        idx0[...] = pos0
        pos1 = _vgather(off_vec, e1v[pl.ds(cb, _L)]) + r1v[pl.ds(cb, _L)]
        idx1[...] = pos1
        pltpu.async_copy(ys_hbm.at[idx0], av, sem)
        pltpu.async_copy(ys_hbm.at[idx1], bv, sem)
        pltpu.async_copy(x_hbm.at[pl.ds(t0 + cb, CH)], xv, sem)

    def drain(b):
        xv, av, bv, idx0, idx1, sem = bufs[b]
        pltpu.make_async_copy(ys_hbm.at[idx0], av, sem).wait()
        pltpu.make_async_copy(ys_hbm.at[idx1], bv, sem).wait()
        pltpu.make_async_copy(x_hbm.at[pl.ds(t0, CH)], xv, sem).wait()

    def compute(c, b):
        xv, av, bv, idx0, idx1, sem = bufs[b]
        cb = pl.multiple_of(c * CH, CH)
        w0c = w0v[pl.ds(cb, _L)]
        w1c = w1v[pl.ds(cb, _L)]
        for j in range(CH):
            jv = jnp.full((_L,), j, jnp.int32)
            wa = _vgather(w0c, jv)
            wb = _vgather(w1c, jv)

            UN = 8

            def vg(v, cc, j=j, wa=wa, wb=wb, xv=xv, av=av, bv=bv):
                for u in range(UN):
                    sl = pl.ds(pl.multiple_of(v * (_L * UN), _L) + u * _L, _L)
                    xv[j, sl] = xv[j, sl] + wa * av[j, sl] + wb * bv[j, sl]
                return cc

            lax.fori_loop(0, D // (_L * UN), vg, 0)
        pltpu.sync_copy(xv, y_hbm.at[pl.ds(t0 + cb, CH)])

    fire(0, 0)

    def outer(g, carry):
        c0 = pl.multiple_of(g * 2, 2)
        fire(c0 + 1, 1)
        drain(0)
        compute(c0, 0)

        @pl.when(g < nch // 2 - 1)
        def _():
            fire(c0 + 2, 0)

        drain(1)
        compute(c0 + 1, 1)
        return carry

    lax.fori_loop(0, nch // 2, outer, 0)


def kernel(x, Wr, br, W1, b1, W2, b2):
    T, D = x.shape
    E = Wr.shape[1]
    H = W1.shape[2]
    BT_R = 1024
    nblk = T // BT_R

    outs = pl.pallas_call(
        _router_body,
        grid=(nblk,),
        in_specs=[
            pl.BlockSpec((BT_R, D), lambda t: (t, 0)),
            pl.BlockSpec((D, E), lambda t: (0, 0)),
            pl.BlockSpec((E,), lambda t: (0,)),
        ],
        out_specs=[
            pl.BlockSpec((1, 1, BT_R), lambda t: (t, 0, 0)),
            pl.BlockSpec((1, 1, BT_R), lambda t: (t, 0, 0)),
            pl.BlockSpec((1, 1, BT_R), lambda t: (t, 0, 0)),
            pl.BlockSpec((1, 1, BT_R), lambda t: (t, 0, 0)),
            pl.BlockSpec((1, 1, BT_R), lambda t: (t, 0, 0)),
            pl.BlockSpec((1, 1, BT_R), lambda t: (t, 0, 0)),
            pl.BlockSpec((1, E), lambda t: (0, 0)),
        ],
        out_shape=[
            jax.ShapeDtypeStruct((nblk, 1, BT_R), jnp.int32),
            jax.ShapeDtypeStruct((nblk, 1, BT_R), jnp.int32),
            jax.ShapeDtypeStruct((nblk, 1, BT_R), jnp.int32),
            jax.ShapeDtypeStruct((nblk, 1, BT_R), jnp.int32),
            jax.ShapeDtypeStruct((nblk, 1, BT_R), jnp.float32),
            jax.ShapeDtypeStruct((nblk, 1, BT_R), jnp.float32),
            jax.ShapeDtypeStruct((1, E), jnp.int32),
        ],
        scratch_shapes=[pltpu.VMEM((8, E), jnp.float32)],
    )(x, Wr, br)
    e0, e1, r0, r1, w0, w1, counts = outs
    e0 = e0.reshape(T)
    e1 = e1.reshape(T)
    r0 = r0.reshape(T)
    r1 = r1.reshape(T)
    w0 = w0.reshape(T)
    w1 = w1.reshape(T)
    counts = counts.reshape(E)

    padded = ((counts + TILE_R - 1) // TILE_R) * TILE_R
    ends = jnp.cumsum(padded)
    off = (ends - padded).astype(jnp.int32)
    PADDED = T * 2 + E * TILE_R
    NT = PADDED // TILE_R
    tile_starts = jnp.arange(NT, dtype=jnp.int32) * TILE_R
    gids = jnp.clip(jnp.searchsorted(ends, tile_starts, side="right"),
                    0, E - 1).astype(jnp.int32)
    n_used = (ends[E - 1] // TILE_R).astype(jnp.int32)
    gids = jnp.concatenate([gids, n_used[None]])

    mesh = plsc.VectorSubcoreMesh(core_axis_name="c", subcore_axis_name="s")
    CH_A = 32
    dispatch = pl.kernel(
        functools.partial(_sc_dispatch_body, T, D, E, CH_A),
        mesh=mesh,
        out_type=jax.ShapeDtypeStruct((PADDED, D), jnp.float32),
        scratch_types=[
            pltpu.VMEM((CH_A, D), jnp.float32),
            pltpu.VMEM((CH_A, D), jnp.float32),
            pltpu.VMEM((T // _NW,), jnp.int32),
            pltpu.VMEM((T // _NW,), jnp.int32),
            pltpu.VMEM((T // _NW,), jnp.int32),
            pltpu.VMEM((T // _NW,), jnp.int32),
            pltpu.VMEM((E,), jnp.int32),
            pltpu.VMEM((CH_A,), jnp.int32),
            pltpu.VMEM((CH_A,), jnp.int32),
            pltpu.VMEM((CH_A,), jnp.int32),
            pltpu.VMEM((CH_A,), jnp.int32),
            pltpu.SemaphoreType.DMA,
            pltpu.SemaphoreType.DMA,
            pltpu.SemaphoreType.DMA,
            pltpu.SemaphoreType.DMA,
        ],
    )
    xs = dispatch(x, e0, e1, r0, r1, off)

    b1r = b1.reshape(E, 1, H)
    b2r = b2.reshape(E, 1, D)
    grid_spec = pltpu.PrefetchScalarGridSpec(
        num_scalar_prefetch=1,
        grid=(NT,),
        in_specs=[
            pl.BlockSpec((TILE_R, D), lambda i, g: (i, 0)),
            pl.BlockSpec((1, D, H), lambda i, g: (g[i], 0, 0)),
            pl.BlockSpec((1, 1, H), lambda i, g: (g[i], 0, 0)),
            pl.BlockSpec((1, H, D), lambda i, g: (g[i], 0, 0)),
            pl.BlockSpec((1, 1, D), lambda i, g: (g[i], 0, 0)),
        ],
        out_specs=pl.BlockSpec((TILE_R, D), lambda i, g: (i, 0)),
    )
    ys = pl.pallas_call(
        _mlp_body,
        grid_spec=grid_spec,
        out_shape=jax.ShapeDtypeStruct((PADDED, D), jnp.float32),
    )(gids, xs, W1, b1r, W2, b2r)

    CH_B = 16
    combine = pl.kernel(
        functools.partial(_sc_combine_body, T, D, E, CH_B),
        mesh=mesh,
        out_type=jax.ShapeDtypeStruct((T, D), jnp.float32),
        scratch_types=[
            pltpu.VMEM((CH_B, D), jnp.float32),
            pltpu.VMEM((CH_B, D), jnp.float32),
            pltpu.VMEM((CH_B, D), jnp.float32),
            pltpu.VMEM((CH_B, D), jnp.float32),
            pltpu.VMEM((CH_B, D), jnp.float32),
            pltpu.VMEM((CH_B, D), jnp.float32),
            pltpu.VMEM((T // _NW,), jnp.int32),
            pltpu.VMEM((T // _NW,), jnp.int32),
            pltpu.VMEM((T // _NW,), jnp.int32),
            pltpu.VMEM((T // _NW,), jnp.int32),
            pltpu.VMEM((T // _NW,), jnp.float32),
            pltpu.VMEM((T // _NW,), jnp.float32),
            pltpu.VMEM((E,), jnp.int32),
            pltpu.VMEM((_L,), jnp.int32),
            pltpu.VMEM((_L,), jnp.int32),
            pltpu.VMEM((_L,), jnp.int32),
            pltpu.VMEM((_L,), jnp.int32),
            pltpu.SemaphoreType.DMA,
            pltpu.SemaphoreType.DMA,
        ],
    )
    y = combine(x, ys, e0, e1, r0, r1, w0, w1, off)
    return y
